# baseline (speedup 1.0000x reference)
import functools

import jax
import jax.numpy as jnp
from jax import lax
from jax.experimental import pallas as pl
from jax.experimental.pallas import tpu as pltpu
from jax.experimental.pallas import tpu_sc as plsc

_R2 = 0.18 ** 2
_K = 32
_EPS = 1e-5
_N = 1024
_B = 4
_D = 128
_L = 16
_NW = 32
_RPW = (_B * _N) // _NW
_CH = 8

_PREC = jax.lax.Precision.HIGHEST


def _ln(h, g, b):
    mu = jnp.mean(h, axis=-1, keepdims=True)
    var = jnp.mean((h - mu) ** 2, axis=-1, keepdims=True)
    return (h - mu) * jax.lax.rsqrt(var + _EPS) * g + b


def _k1_body(f_ref, w1_ref, b1_ref, g1_ref, be1_ref, wzy_ref, bzy_ref, zy_ref):
    h = jnp.dot(f_ref[0], w1_ref[...], preferred_element_type=jnp.float32,
                precision=_PREC)
    h = h + b1_ref[...][None, :]
    x = jnp.maximum(_ln(h, g1_ref[...][None, :], be1_ref[...][None, :]), 0.0)
    zy = jnp.dot(x, wzy_ref[...], preferred_element_type=jnp.float32,
                 precision=_PREC)
    zy_ref[0] = zy + bzy_ref[...][None, :]


_GDN = lax.GatherDimensionNumbers(
    offset_dims=(), collapsed_slice_dims=(0,), start_index_map=(0,))


def _dyn_gather(x, idx):
    return lax.gather(x, idx[:, None], _GDN, slice_sizes=(1,),
                      mode=lax.GatherScatterMode.PROMISE_IN_BOUNDS)


def _rsqrt_sc(x):
    i = lax.bitcast_convert_type(x, jnp.int32)
    i = 0x5F3759DF - (i >> 1)
    r = lax.bitcast_convert_type(i, jnp.float32)
    for _ in range(3):
        r = r * (1.5 - 0.5 * x * r * r)
    return r


def _sc_body(dis_hbm, z_hbm, y_hbm, g2_hbm, be2_hbm, out_hbm,
             dis_v, z_v, rows_v, out_v, gidx_v, idxbuf_v, idxfin_v,
             g2_v, be2_v, sem):
    wid = lax.axis_index("s") * 2 + lax.axis_index("c")
    base = wid * _RPW
    bbase = (wid // 8) * _N

    pltpu.sync_copy(g2_hbm, g2_v)
    pltpu.sync_copy(be2_hbm, be2_v)
    pltpu.sync_copy(z_hbm.at[pl.ds(base, _RPW)], z_v)

    lane = jnp.arange(_L, dtype=jnp.int32)
    for c in range(_N // _L):
        gidx_v[pl.ds(c * _L, _L)] = lane + (bbase + c * _L)

    g2h = [g2_v[pl.ds(h * _L, _L)] for h in range(_D // _L)]
    be2h = [be2_v[pl.ds(h * _L, _L)] for h in range(_D // _L)]
    zero16 = jnp.zeros((_L,), jnp.int32)
    lane15 = jnp.full((_L,), _L - 1, jnp.int32)

    def point_body(p, ci):
        sentv = _dyn_gather(gidx_v[pl.ds(_N - _L, _L)], lane15)
        idxbuf_v[pl.ds(0, _L)] = sentv
        off = zero16
        for c in range(_N // _L):
            d = dis_v[p, pl.ds(c * _L, _L)]
            m = d <= _R2
            mi = m.astype(jnp.int32)
            cs = plsc.cumsum(mi)
            pos = off + cs - mi
            msk = m & (pos < _K)
            plsc.store_scatter(idxbuf_v, [pos], gidx_v[pl.ds(c * _L, _L)],
                               mask=msk)
            off = off + _dyn_gather(cs, lane15)
        kcnt = jnp.minimum(off, _K)

        first = _dyn_gather(idxbuf_v[pl.ds(0, _L)], zero16)
        copies = []
        for h in range(_K // _L):
            ki = lane + h * _L
            gath = plsc.load_gather(idxbuf_v, [ki])
            sm = (ki < kcnt).astype(jnp.int32)
            vals = gath * sm + first * (1 - sm)
            copies.append(pltpu.async_copy(
                y_hbm.at[vals], rows_v.at[pl.ds(h * _L, _L)], sem))
        for cp in copies:
            cp.wait()

        row = ci * _CH + p
        zh = [z_v[row, pl.ds(h * _L, _L)] for h in range(_D // _L)]

        def nb(k, carry):
            accs, accm = carry
            dh = [zh[h] - rows_v[k, pl.ds(h * _L, _L)]
                  for h in range(_D // _L)]
            t = dh[0]
            q = dh[0] * dh[0]
            for h in range(1, _D // _L):
                t = t + dh[h]
                q = q + dh[h] * dh[h]
            s = jnp.sum(t)
            sq = jnp.sum(q)
            mu = s * (1.0 / _D)
            var = sq * (1.0 / _D) - mu * mu
            inv = _rsqrt_sc(var + _EPS)
            new_s = []
            new_m = []
            for h in range(_D // _L):
                v = jnp.maximum((dh[h] - mu) * (inv * g2h[h]) + be2h[h], 0.0)
                new_s.append(accs[h] + v)
                new_m.append(jnp.maximum(accm[h], v))
            return tuple(new_s), tuple(new_m)

        z16 = jnp.zeros((_L,), jnp.float32)
        accs, accm = lax.fori_loop(
            0, _K, nb,
            (tuple(z16 for _ in range(_D // _L)),
             tuple(z16 for _ in range(_D // _L))))
        for h in range(_D // _L):
            out_v[row, pl.ds(h * _L, _L)] = accs[h] * (1.0 / _K)
        for h in range(_D // _L):
            out_v[row, pl.ds(_D + h * _L, _L)] = accm[h]

    def chunk_body(ci, _):
        pltpu.sync_copy(dis_hbm.at[pl.ds(base + ci * _CH, _CH)], dis_v)
        lax.fori_loop(0, _CH, lambda p, __: (point_body(p, ci), 0)[1], 0)
        return 0

    lax.fori_loop(0, _RPW // _CH, chunk_body, 0)
    pltpu.sync_copy(out_v, out_hbm.at[pl.ds(base, _RPW)])


@jax.jit
def kernel(features, coordinates, dis_mats, W1, b1, g1, be1, W2, b2, g2, be2):
    del coordinates
    B, N, init_dim = features.shape
    dim = W1.shape[1]
    half = W2.shape[1]

    w2a, w2b = W2[:dim], W2[dim:]
    wzy = jnp.concatenate([w2a + w2b, w2b], axis=1)
    bzy = jnp.concatenate([b2, jnp.zeros((half,), b2.dtype)])

    zy = pl.pallas_call(
        _k1_body,
        grid=(B,),
        in_specs=[
            pl.BlockSpec((1, N, init_dim), lambda b: (b, 0, 0)),
            pl.BlockSpec((init_dim, dim), lambda b: (0, 0)),
            pl.BlockSpec((dim,), lambda b: (0,)),
            pl.BlockSpec((dim,), lambda b: (0,)),
            pl.BlockSpec((dim,), lambda b: (0,)),
            pl.BlockSpec((dim, 2 * half), lambda b: (0, 0)),
            pl.BlockSpec((2 * half,), lambda b: (0,)),
        ],
        out_specs=pl.BlockSpec((1, N, 2 * half), lambda b: (b, 0, 0)),
        out_shape=jax.ShapeDtypeStruct((B, N, 2 * half), jnp.float32),
        compiler_params=pltpu.CompilerParams(
            dimension_semantics=("parallel",)),
    )(features, W1, b1, g1, be1, wzy, bzy)

    z2 = zy[..., :half].reshape(B * N, half)
    y2 = zy[..., half:].reshape(B * N, half)
    dis2 = dis_mats.reshape(B * N, N)

    mesh = plsc.VectorSubcoreMesh(core_axis_name="c", subcore_axis_name="s")
    sc = functools.partial(
        pl.kernel, mesh=mesh,
        out_type=jax.ShapeDtypeStruct((B * N, 2 * half), jnp.float32),
        compiler_params=pltpu.CompilerParams(needs_layout_passes=False),
        scratch_types=[
            pltpu.VMEM((_CH, N), jnp.float32),
            pltpu.VMEM((_RPW, half), jnp.float32),
            pltpu.VMEM((_K, half), jnp.float32),
            pltpu.VMEM((_RPW, 2 * half), jnp.float32),
            pltpu.VMEM((N,), jnp.int32),
            pltpu.VMEM((_K,), jnp.int32),
            pltpu.VMEM((_K,), jnp.int32),
            pltpu.VMEM((half,), jnp.float32),
            pltpu.VMEM((half,), jnp.float32),
            pltpu.SemaphoreType.DMA,
        ],
    )(_sc_body)
    out2 = sc(dis2, z2, y2, g2, be2)

    return out2.reshape(B, N, 2 * half)


# speedup vs baseline: 1.0910x; 1.0910x over previous
import functools

import jax
import jax.numpy as jnp
from jax import lax
from jax.experimental import pallas as pl
from jax.experimental.pallas import tpu as pltpu
from jax.experimental.pallas import tpu_sc as plsc

_R2 = 0.18 ** 2
_K = 32
_EPS = 1e-5
_N = 1024
_B = 4
_D = 128
_L = 16
_NW = 32
_RPW = (_B * _N) // _NW
_CH = 8
_P3 = 128

_PREC = jax.lax.Precision.HIGHEST


def _ln(h, g, b):
    mu = jnp.mean(h, axis=-1, keepdims=True)
    var = jnp.mean((h - mu) ** 2, axis=-1, keepdims=True)
    return (h - mu) * jax.lax.rsqrt(var + _EPS) * g + b


def _k1_body(f_ref, w1_ref, b1_ref, g1_ref, be1_ref, wzy_ref, bzy_ref, zy_ref):
    h = jnp.dot(f_ref[0], w1_ref[...], preferred_element_type=jnp.float32,
                precision=_PREC)
    h = h + b1_ref[...][None, :]
    x = jnp.maximum(_ln(h, g1_ref[...][None, :], be1_ref[...][None, :]), 0.0)
    zy = jnp.dot(x, wzy_ref[...], preferred_element_type=jnp.float32,
                 precision=_PREC)
    zy_ref[0] = zy + bzy_ref[...][None, :]


def _k3_body(z_ref, gath_ref, g2_ref, be2_ref, out_ref):
    d = z_ref[...][:, None, :] - gath_ref[...].reshape(_P3, _K, _D)
    mu = jnp.mean(d, axis=-1, keepdims=True)
    var = jnp.mean((d - mu) ** 2, axis=-1, keepdims=True)
    v = (d - mu) * lax.rsqrt(var + _EPS)
    v = jnp.maximum(v * g2_ref[...][None, None, :]
                    + be2_ref[...][None, None, :], 0.0)
    out_ref[...] = jnp.concatenate(
        [jnp.mean(v, axis=1), jnp.max(v, axis=1)], axis=-1)


_GDN = lax.GatherDimensionNumbers(
    offset_dims=(), collapsed_slice_dims=(0,), start_index_map=(0,))


def _dyn_gather(x, idx):
    return lax.gather(x, idx[:, None], _GDN, slice_sizes=(1,),
                      mode=lax.GatherScatterMode.PROMISE_IN_BOUNDS)


def _sc_body(dis_hbm, y_hbm, gath_hbm,
             dis_v, rows_v, gidx_v, idxbuf_v, sem):
    wid = lax.axis_index("s") * 2 + lax.axis_index("c")
    base = wid * _RPW
    bbase = (wid // 8) * _N

    lane = jnp.arange(_L, dtype=jnp.int32)
    for c in range(_N // _L):
        gidx_v[pl.ds(c * _L, _L)] = lane + (bbase + c * _L)

    zero16 = jnp.zeros((_L,), jnp.int32)
    lane15 = jnp.full((_L,), _L - 1, jnp.int32)

    def point_body(p, _):
        sentv = _dyn_gather(gidx_v[pl.ds(_N - _L, _L)], lane15)
        idxbuf_v[pl.ds(0, _L)] = sentv
        off = zero16
        for c in range(_N // _L):
            d = dis_v[p, pl.ds(c * _L, _L)]
            m = d <= _R2
            mi = m.astype(jnp.int32)
            cs = plsc.cumsum(mi)
            pos = off + cs - mi
            msk = m & (pos < _K)
            plsc.store_scatter(idxbuf_v, [pos], gidx_v[pl.ds(c * _L, _L)],
                               mask=msk)
            off = off + _dyn_gather(cs, lane15)
        kcnt = jnp.minimum(off, _K)

        first = _dyn_gather(idxbuf_v[pl.ds(0, _L)], zero16)
        copies = []
        for h in range(_K // _L):
            ki = lane + h * _L
            gath = plsc.load_gather(idxbuf_v, [ki])
            sm = (ki < kcnt).astype(jnp.int32)
            vals = gath * sm + first * (1 - sm)
            copies.append(pltpu.async_copy(
                y_hbm.at[vals], rows_v.at[pl.ds(p * _K + h * _L, _L)], sem))
        for cp in copies:
            cp.wait()
        return 0

    def chunk_body(ci, _):
        pltpu.sync_copy(dis_hbm.at[pl.ds(base + ci * _CH, _CH)], dis_v)
        lax.fori_loop(0, _CH, point_body, 0)
        pltpu.sync_copy(rows_v,
                        gath_hbm.at[pl.ds((base + ci * _CH) * _K, _CH * _K)])
        return 0

    lax.fori_loop(0, _RPW // _CH, chunk_body, 0)


@jax.jit
def kernel(features, coordinates, dis_mats, W1, b1, g1, be1, W2, b2, g2, be2):
    del coordinates
    B, N, init_dim = features.shape
    dim = W1.shape[1]
    half = W2.shape[1]

    w2a, w2b = W2[:dim], W2[dim:]
    wzy = jnp.concatenate([w2a + w2b, w2b], axis=1)
    bzy = jnp.concatenate([b2, jnp.zeros((half,), b2.dtype)])

    zy = pl.pallas_call(
        _k1_body,
        grid=(B,),
        in_specs=[
            pl.BlockSpec((1, N, init_dim), lambda b: (b, 0, 0)),
            pl.BlockSpec((init_dim, dim), lambda b: (0, 0)),
            pl.BlockSpec((dim,), lambda b: (0,)),
            pl.BlockSpec((dim,), lambda b: (0,)),
            pl.BlockSpec((dim,), lambda b: (0,)),
            pl.BlockSpec((dim, 2 * half), lambda b: (0, 0)),
            pl.BlockSpec((2 * half,), lambda b: (0,)),
        ],
        out_specs=pl.BlockSpec((1, N, 2 * half), lambda b: (b, 0, 0)),
        out_shape=jax.ShapeDtypeStruct((B, N, 2 * half), jnp.float32),
        compiler_params=pltpu.CompilerParams(
            dimension_semantics=("parallel",)),
    )(features, W1, b1, g1, be1, wzy, bzy)

    z2 = zy[..., :half].reshape(B * N, half)
    y2 = zy[..., half:].reshape(B * N, half)
    dis2 = dis_mats.reshape(B * N, N)

    mesh = plsc.VectorSubcoreMesh(core_axis_name="c", subcore_axis_name="s")
    sc = functools.partial(
        pl.kernel, mesh=mesh,
        out_type=jax.ShapeDtypeStruct((B * N * _K, half), jnp.float32),
        compiler_params=pltpu.CompilerParams(needs_layout_passes=False),
        scratch_types=[
            pltpu.VMEM((_CH, N), jnp.float32),
            pltpu.VMEM((_CH * _K, half), jnp.float32),
            pltpu.VMEM((N,), jnp.int32),
            pltpu.VMEM((_K,), jnp.int32),
            pltpu.SemaphoreType.DMA,
        ],
    )(_sc_body)
    gath = sc(dis2, y2)

    out2 = pl.pallas_call(
        _k3_body,
        grid=(B * N // _P3,),
        in_specs=[
            pl.BlockSpec((_P3, half), lambda b: (b, 0)),
            pl.BlockSpec((_P3 * _K, half), lambda b: (b, 0)),
            pl.BlockSpec((half,), lambda b: (0,)),
            pl.BlockSpec((half,), lambda b: (0,)),
        ],
        out_specs=pl.BlockSpec((_P3, 2 * half), lambda b: (b, 0)),
        out_shape=jax.ShapeDtypeStruct((B * N, 2 * half), jnp.float32),
        compiler_params=pltpu.CompilerParams(
            dimension_semantics=("parallel",)),
    )(z2, gath, g2, be2)

    return out2.reshape(B, N, 2 * half)


# speedup vs baseline: 1.1018x; 1.0099x over previous
import functools

import jax
import jax.numpy as jnp
from jax import lax
from jax.experimental import pallas as pl
from jax.experimental.pallas import tpu as pltpu
from jax.experimental.pallas import tpu_sc as plsc

_R2 = 0.18 ** 2
_K = 32
_EPS = 1e-5
_N = 1024
_B = 4
_D = 128
_L = 16
_NW = 32
_RPW = (_B * _N) // _NW
_CH = 8
_P3 = 128

_PREC = jax.lax.Precision.HIGHEST


def _ln(h, g, b):
    mu = jnp.mean(h, axis=-1, keepdims=True)
    var = jnp.mean((h - mu) ** 2, axis=-1, keepdims=True)
    return (h - mu) * jax.lax.rsqrt(var + _EPS) * g + b


def _k1_body(f_ref, w1_ref, b1_ref, g1_ref, be1_ref, wzy_ref, bzy_ref, zy_ref):
    h = jnp.dot(f_ref[0], w1_ref[...], preferred_element_type=jnp.float32,
                precision=_PREC)
    h = h + b1_ref[...][None, :]
    x = jnp.maximum(_ln(h, g1_ref[...][None, :], be1_ref[...][None, :]), 0.0)
    zy = jnp.dot(x, wzy_ref[...], preferred_element_type=jnp.float32,
                 precision=_PREC)
    zy_ref[0] = zy + bzy_ref[...][None, :]


def _k3_body(z_ref, gath_ref, g2_ref, be2_ref, out_ref):
    d = z_ref[...][:, None, :] - gath_ref[...].reshape(_P3, _K, _D)
    mu = jnp.mean(d, axis=-1, keepdims=True)
    var = jnp.mean((d - mu) ** 2, axis=-1, keepdims=True)
    v = (d - mu) * lax.rsqrt(var + _EPS)
    v = jnp.maximum(v * g2_ref[...][None, None, :]
                    + be2_ref[...][None, None, :], 0.0)
    out_ref[...] = jnp.concatenate(
        [jnp.mean(v, axis=1), jnp.max(v, axis=1)], axis=-1)


_GDN = lax.GatherDimensionNumbers(
    offset_dims=(), collapsed_slice_dims=(0,), start_index_map=(0,))


def _dyn_gather(x, idx):
    return lax.gather(x, idx[:, None], _GDN, slice_sizes=(1,),
                      mode=lax.GatherScatterMode.PROMISE_IN_BOUNDS)


def _sc_body(dis_hbm, y_hbm, gath_hbm,
             dis_v, rows_v, gidx_v, idxbuf_v, sem):
    wid = lax.axis_index("s") * 2 + lax.axis_index("c")
    base = wid * _RPW
    bbase = (wid // 8) * _N

    lane = jnp.arange(_L, dtype=jnp.int32)
    for c in range(_N // _L):
        gidx_v[pl.ds(c * _L, _L)] = lane + (bbase + c * _L)

    zero16 = jnp.zeros((_L,), jnp.int32)
    lane15 = jnp.full((_L,), _L - 1, jnp.int32)
    sentv = _dyn_gather(gidx_v[pl.ds(_N - _L, _L)], lane15)

    def chunk_body(ci, _):
        pltpu.sync_copy(dis_hbm.at[pl.ds(base + ci * _CH, _CH)], dis_v)
        copies = []
        for p in range(_CH):
            idxbuf_v[pl.ds(0, _L)] = sentv
            off = zero16
            for c in range(_N // _L):
                d = dis_v[p, pl.ds(c * _L, _L)]
                m = d <= _R2
                mi = m.astype(jnp.int32)
                cs = plsc.cumsum(mi)
                pos = off + cs - mi
                msk = m & (pos < _K)
                plsc.store_scatter(idxbuf_v, [pos], gidx_v[pl.ds(c * _L, _L)],
                                   mask=msk)
                off = off + _dyn_gather(cs, lane15)
            kcnt = jnp.minimum(off, _K)

            first = _dyn_gather(idxbuf_v[pl.ds(0, _L)], zero16)
            for h in range(_K // _L):
                ki = lane + h * _L
                gath = plsc.load_gather(idxbuf_v, [ki])
                sm = (ki < kcnt).astype(jnp.int32)
                vals = gath * sm + first * (1 - sm)
                copies.append(pltpu.async_copy(
                    y_hbm.at[vals], rows_v.at[pl.ds(p * _K + h * _L, _L)],
                    sem))
        for cp in copies:
            cp.wait()
        pltpu.sync_copy(rows_v,
                        gath_hbm.at[pl.ds((base + ci * _CH) * _K, _CH * _K)])
        return 0

    lax.fori_loop(0, _RPW // _CH, chunk_body, 0)


@jax.jit
def kernel(features, coordinates, dis_mats, W1, b1, g1, be1, W2, b2, g2, be2):
    del coordinates
    B, N, init_dim = features.shape
    dim = W1.shape[1]
    half = W2.shape[1]

    w2a, w2b = W2[:dim], W2[dim:]
    wzy = jnp.concatenate([w2a + w2b, w2b], axis=1)
    bzy = jnp.concatenate([b2, jnp.zeros((half,), b2.dtype)])

    zy = pl.pallas_call(
        _k1_body,
        grid=(B,),
        in_specs=[
            pl.BlockSpec((1, N, init_dim), lambda b: (b, 0, 0)),
            pl.BlockSpec((init_dim, dim), lambda b: (0, 0)),
            pl.BlockSpec((dim,), lambda b: (0,)),
            pl.BlockSpec((dim,), lambda b: (0,)),
            pl.BlockSpec((dim,), lambda b: (0,)),
            pl.BlockSpec((dim, 2 * half), lambda b: (0, 0)),
            pl.BlockSpec((2 * half,), lambda b: (0,)),
        ],
        out_specs=pl.BlockSpec((1, N, 2 * half), lambda b: (b, 0, 0)),
        out_shape=jax.ShapeDtypeStruct((B, N, 2 * half), jnp.float32),
        compiler_params=pltpu.CompilerParams(
            dimension_semantics=("parallel",)),
    )(features, W1, b1, g1, be1, wzy, bzy)

    z2 = zy[..., :half].reshape(B * N, half)
    y2 = zy[..., half:].reshape(B * N, half)
    dis2 = dis_mats.reshape(B * N, N)

    mesh = plsc.VectorSubcoreMesh(core_axis_name="c", subcore_axis_name="s")
    sc = functools.partial(
        pl.kernel, mesh=mesh,
        out_type=jax.ShapeDtypeStruct((B * N * _K, half), jnp.float32),
        compiler_params=pltpu.CompilerParams(needs_layout_passes=False),
        scratch_types=[
            pltpu.VMEM((_CH, N), jnp.float32),
            pltpu.VMEM((_CH * _K, half), jnp.float32),
            pltpu.VMEM((N,), jnp.int32),
            pltpu.VMEM((_K,), jnp.int32),
            pltpu.SemaphoreType.DMA,
        ],
    )(_sc_body)
    gath = sc(dis2, y2)

    out2 = pl.pallas_call(
        _k3_body,
        grid=(B * N // _P3,),
        in_specs=[
            pl.BlockSpec((_P3, half), lambda b: (b, 0)),
            pl.BlockSpec((_P3 * _K, half), lambda b: (b, 0)),
            pl.BlockSpec((half,), lambda b: (0,)),
            pl.BlockSpec((half,), lambda b: (0,)),
        ],
        out_specs=pl.BlockSpec((_P3, 2 * half), lambda b: (b, 0)),
        out_shape=jax.ShapeDtypeStruct((B * N, 2 * half), jnp.float32),
        compiler_params=pltpu.CompilerParams(
            dimension_semantics=("parallel",)),
    )(z2, gath, g2, be2)

    return out2.reshape(B, N, 2 * half)
